# Initial kernel scaffold; baseline (speedup 1.0000x reference)
import jax
import jax.numpy as jnp
import numpy as np
from jax.experimental import pallas as pl

NT = 50000
NM = 50000
D_IN = 128
D = 64
L = 4
OUT = 8
E = 800000

_ROWS = 2048


def _mm_bias_kernel(x_ref, w_ref, b_ref, o_ref):
    o_ref[...] = (
        jnp.dot(x_ref[...], w_ref[...], preferred_element_type=jnp.float32)
        + b_ref[...]
    )


def _mm_bias(x, w, b):
    n, kdim = x.shape
    m = w.shape[1]
    grid = (pl.cdiv(n, _ROWS),)
    return pl.pallas_call(
        _mm_bias_kernel,
        grid=grid,
        in_specs=[
            pl.BlockSpec((_ROWS, kdim), lambda i: (i, 0)),
            pl.BlockSpec((kdim, m), lambda i: (0, 0)),
            pl.BlockSpec((m,), lambda i: (0,)),
        ],
        out_specs=pl.BlockSpec((_ROWS, m), lambda i: (i, 0)),
        out_shape=jax.ShapeDtypeStruct((n, m), jnp.float32),
    )(x, w, b)


def _proj_kernel(x_ref, qw_ref, qb_ref, kw_ref, kb_ref, vw_ref, vb_ref,
                 ra_ref, rm_ref, q_ref, k_ref, v_ref):
    wk = jnp.dot(kw_ref[...], ra_ref[...], preferred_element_type=jnp.float32)
    wv = jnp.dot(vw_ref[...], rm_ref[...], preferred_element_type=jnp.float32)
    bk = jnp.dot(kb_ref[...].reshape(1, -1), ra_ref[...],
                 preferred_element_type=jnp.float32)
    bv = jnp.dot(vb_ref[...].reshape(1, -1), rm_ref[...],
                 preferred_element_type=jnp.float32)
    x = x_ref[...]
    q_ref[...] = jnp.dot(x, qw_ref[...], preferred_element_type=jnp.float32) + qb_ref[...]
    k_ref[...] = jnp.dot(x, wk, preferred_element_type=jnp.float32) + bk
    v_ref[...] = jnp.dot(x, wv, preferred_element_type=jnp.float32) + bv


def _proj(x, qw, qb, kw, kb, vw, vb, ra, rm):
    n = x.shape[0]
    grid = (pl.cdiv(n, _ROWS),)
    full = lambda i: (0, 0)
    vec = lambda i: (0,)
    row = lambda i: (i, 0)
    outs = [jax.ShapeDtypeStruct((n, D), jnp.float32)] * 3
    return pl.pallas_call(
        _proj_kernel,
        grid=grid,
        in_specs=[
            pl.BlockSpec((_ROWS, D), row),
            pl.BlockSpec((D, D), full), pl.BlockSpec((D,), vec),
            pl.BlockSpec((D, D), full), pl.BlockSpec((D,), vec),
            pl.BlockSpec((D, D), full), pl.BlockSpec((D,), vec),
            pl.BlockSpec((D, D), full), pl.BlockSpec((D, D), full),
        ],
        out_specs=[pl.BlockSpec((_ROWS, D), row)] * 3,
        out_shape=outs,
    )(x, qw, qb, kw, kb, vw, vb, ra, rm)


def _update_kernel(agg_ref, x_ref, aw_ref, ab_ref, skip_ref, o_ref):
    beta = jax.nn.sigmoid(skip_ref[0])
    o = (
        jnp.dot(jax.nn.gelu(agg_ref[...]), aw_ref[...],
                preferred_element_type=jnp.float32)
        + ab_ref[...]
    )
    o_ref[...] = jax.nn.relu(beta * o + (1.0 - beta) * x_ref[...])


def _update(agg, x, aw, ab, skip_scalar):
    n = agg.shape[0]
    grid = (pl.cdiv(n, _ROWS),)
    return pl.pallas_call(
        _update_kernel,
        grid=grid,
        in_specs=[
            pl.BlockSpec((_ROWS, D), lambda i: (i, 0)),
            pl.BlockSpec((_ROWS, D), lambda i: (i, 0)),
            pl.BlockSpec((D, D), lambda i: (0, 0)),
            pl.BlockSpec((D,), lambda i: (0,)),
            pl.BlockSpec((1,), lambda i: (0,)),
        ],
        out_specs=pl.BlockSpec((_ROWS, D), lambda i: (i, 0)),
        out_shape=jax.ShapeDtypeStruct((n, D), jnp.float32),
    )(agg, x, aw, ab, skip_scalar)


def _edge_stage(k_rel, q_dst, v_rel, src, dst, n_dst, relp):
    scale = 1.0 / np.sqrt(D)
    a = jnp.sum(k_rel[src] * q_dst[dst], axis=-1) * (relp * scale)
    w = jnp.exp(jnp.clip(a, -60.0, 60.0))
    s = jax.ops.segment_sum(w, dst, num_segments=n_dst)
    num = jax.ops.segment_sum(v_rel[src] * w[:, None], dst, num_segments=n_dst)
    return num / (s[:, None] + 1e-16)


def kernel(x_transaction, merchant_ids, edge_index_tm, edge_index_mt,
           lin_tx_W, lin_tx_b, merch_emb, kW, kb, qW, qb, vW, vb, aW, ab,
           relA, relM, relP, skip, outW, outb):
    x0 = _mm_bias(x_transaction, lin_tx_W, lin_tx_b)
    x1 = jnp.take(merch_emb, merchant_ids, axis=0)
    x = {0: x0, 1: x1}
    edges = [(0, 1, 0, edge_index_tm), (1, 0, 1, edge_index_mt)]
    sizes = {0: NT, 1: NM}

    for l in range(L):
        q, krel, vrel = {}, {}, {}
        for (s_t, d_t, r, ei) in edges:
            q[s_t], krel[r], vrel[r] = _proj(
                x[s_t], qW[l, s_t], qb[l, s_t], kW[l, s_t], kb[l, s_t],
                vW[l, s_t], vb[l, s_t], relA[l, r], relM[l, r])
        agg = {}
        for (s_t, d_t, r, ei) in edges:
            agg[d_t] = _edge_stage(krel[r], q[d_t], vrel[r], ei[0], ei[1],
                                   sizes[d_t], relP[l, r])
        newx = {}
        for t in (0, 1):
            newx[t] = _update(agg[t], x[t], aW[l, t], ab[l, t],
                              skip[l, t:t + 1])
        x = newx

    outW_p = jnp.zeros((D, 128), jnp.float32).at[:, :OUT].set(outW)
    outb_p = jnp.zeros((128,), jnp.float32).at[:OUT].set(outb)
    out = _mm_bias(x[0], outW_p, outb_p)
    return out[:, :OUT]


# speedup vs baseline: 10.0627x; 10.0627x over previous
import functools

import jax
import jax.numpy as jnp
import numpy as np
from jax import lax
from jax.experimental import pallas as pl
from jax.experimental.pallas import tpu as pltpu
from jax.experimental.pallas import tpu_sc as plsc

NT = 50000
NM = 50000
D_IN = 128
D = 64
L = 4
OUT = 8
E = 800000

_ROWS = 2048

_NW = 32
_NPAD = 50176
_BDW = 1568
_NSUB = 4
_BD = 392
_EC = 128


def _mm_bias_kernel(x_ref, w_ref, b_ref, o_ref):
    o_ref[...] = (
        jnp.dot(x_ref[...], w_ref[...], preferred_element_type=jnp.float32)
        + b_ref[...]
    )


def _mm_bias(x, w, b, n_out):
    kdim = x.shape[1]
    m = w.shape[1]
    grid = (pl.cdiv(n_out, _ROWS),)
    return pl.pallas_call(
        _mm_bias_kernel,
        grid=grid,
        in_specs=[
            pl.BlockSpec((_ROWS, kdim), lambda i: (i, 0)),
            pl.BlockSpec((kdim, m), lambda i: (0, 0)),
            pl.BlockSpec((m,), lambda i: (0,)),
        ],
        out_specs=pl.BlockSpec((_ROWS, m), lambda i: (i, 0)),
        out_shape=jax.ShapeDtypeStruct((n_out, m), jnp.float32),
    )(x, w, b)


def _proj_kernel(x_ref, qw_ref, qb_ref, kw_ref, kb_ref, vw_ref, vb_ref,
                 ra_ref, rm_ref, q_ref, kv_ref):
    wk = jnp.dot(kw_ref[...], ra_ref[...], preferred_element_type=jnp.float32)
    wv = jnp.dot(vw_ref[...], rm_ref[...], preferred_element_type=jnp.float32)
    bk = jnp.dot(kb_ref[...].reshape(1, -1), ra_ref[...],
                 preferred_element_type=jnp.float32)
    bv = jnp.dot(vb_ref[...].reshape(1, -1), rm_ref[...],
                 preferred_element_type=jnp.float32)
    x = x_ref[...]
    q_ref[...] = jnp.dot(x, qw_ref[...], preferred_element_type=jnp.float32) + qb_ref[...]
    wkv = jnp.concatenate([wk, wv], axis=1)
    bkv = jnp.concatenate([bk, bv], axis=1)
    kv_ref[...] = jnp.dot(x, wkv, preferred_element_type=jnp.float32) + bkv


def _proj(x, qw, qb, kw, kb, vw, vb, ra, rm):
    grid = (pl.cdiv(_NPAD, _ROWS),)
    full = lambda i: (0, 0)
    vec = lambda i: (0,)
    row = lambda i: (i, 0)
    outs = [jax.ShapeDtypeStruct((_NPAD, D), jnp.float32),
            jax.ShapeDtypeStruct((_NPAD, 2 * D), jnp.float32)]
    return pl.pallas_call(
        _proj_kernel,
        grid=grid,
        in_specs=[
            pl.BlockSpec((_ROWS, D), row),
            pl.BlockSpec((D, D), full), pl.BlockSpec((D,), vec),
            pl.BlockSpec((D, D), full), pl.BlockSpec((D,), vec),
            pl.BlockSpec((D, D), full), pl.BlockSpec((D,), vec),
            pl.BlockSpec((D, D), full), pl.BlockSpec((D, D), full),
        ],
        out_specs=[pl.BlockSpec((_ROWS, D), row),
                   pl.BlockSpec((_ROWS, 2 * D), row)],
        out_shape=outs,
    )(x, qw, qb, kw, kb, vw, vb, ra, rm)


def _update_kernel(agg_ref, x_ref, aw_ref, ab_ref, skip_ref, o_ref):
    beta = jax.nn.sigmoid(skip_ref[0])
    o = (
        jnp.dot(jax.nn.gelu(agg_ref[...]), aw_ref[...],
                preferred_element_type=jnp.float32)
        + ab_ref[...]
    )
    o_ref[...] = jax.nn.relu(beta * o + (1.0 - beta) * x_ref[...])


def _update(agg, x, aw, ab, skip_scalar):
    n = agg.shape[0]
    grid = (pl.cdiv(n, _ROWS),)
    return pl.pallas_call(
        _update_kernel,
        grid=grid,
        in_specs=[
            pl.BlockSpec((_ROWS, D), lambda i: (i, 0)),
            pl.BlockSpec((_ROWS, D), lambda i: (i, 0)),
            pl.BlockSpec((D, D), lambda i: (0, 0)),
            pl.BlockSpec((D,), lambda i: (0,)),
            pl.BlockSpec((1,), lambda i: (0,)),
        ],
        out_specs=pl.BlockSpec((_ROWS, D), lambda i: (i, 0)),
        out_shape=jax.ShapeDtypeStruct((n, D), jnp.float32),
    )(agg, x, aw, ab, skip_scalar)


def _edge_sc(kv, q_scaled, src_s, dst_s, rowptr):
    mesh = plsc.VectorSubcoreMesh(core_axis_name="c", subcore_axis_name="s")

    @functools.partial(
        pl.kernel,
        out_type=jax.ShapeDtypeStruct((_NPAD, D), jnp.float32),
        mesh=mesh,
        compiler_params=pltpu.CompilerParams(needs_layout_passes=False),
        scratch_types=[
            pltpu.VMEM((1584,), jnp.int32),
            pltpu.VMEM((_BD, D), jnp.float32),
            pltpu.VMEM((_BD, 80), jnp.float32),
            pltpu.VMEM((_EC,), jnp.int32),
            pltpu.VMEM((_EC + 16,), jnp.int32),
            pltpu.VMEM((_EC, 2 * D), jnp.float32),
        ],
    )
    def ker(kv_h, q_h, src_h, dst_h, rp_h, out_h,
            rp_v, q_v, acc_v, src_v, dst_v, kvv):
        wid = lax.axis_index("s") * 2 + lax.axis_index("c")
        w0 = wid * _BDW
        pltpu.sync_copy(rp_h.at[pl.ds(w0, 1584)], rp_v)

        def _sload(ref, idx):
            return ref[pl.ds(idx, 16)][0]

        @pl.loop(0, _NSUB)
        def _sub(sub):
            d0s = w0 + sub * _BD
            e_lo = _sload(rp_v, sub * _BD)
            e_hi = _sload(rp_v, (sub + 1) * _BD)

            @pl.loop(0, _BD)
            def _zero(dd):
                zero16 = jnp.zeros((16,), jnp.float32)
                for f in range(5):
                    acc_v[dd, pl.ds(f * 16, 16)] = zero16

            pltpu.sync_copy(q_h.at[pl.ds(d0s, _BD)], q_v)

            c0 = (e_lo // 8) * 8
            nch = (e_hi - c0 + _EC - 1) // _EC

            def chunk_body(j, carry):
                base = c0 + j * _EC
                pltpu.sync_copy(src_h.at[pl.ds(base, _EC)], src_v)
                pltpu.sync_copy(dst_h.at[pl.ds(base, _EC)],
                                dst_v.at[pl.ds(0, _EC)])
                pltpu.sync_copy(kv_h.at[src_v], kvv)
                i_lo = jnp.maximum(e_lo - base, 0)
                i_hi = jnp.minimum(e_hi - base, _EC)

                def seg_body(i):
                    d = _sload(dst_v, i) - d0s
                    seg_end = jnp.minimum(
                        _sload(rp_v, d + sub * _BD + 1) - base, i_hi)
                    vq = [q_v[d, pl.ds(f * 16, 16)] for f in range(4)]
                    zero16 = jnp.zeros((16,), jnp.float32)

                    def edge_body(i2, c2):
                        a0, a1, a2, a3, ws = c2
                        dot = kvv[i2, pl.ds(0, 16)] * vq[0]
                        for f in range(1, 4):
                            dot = dot + kvv[i2, pl.ds(f * 16, 16)] * vq[f]
                        a = jnp.minimum(jnp.maximum(jnp.sum(dot), -60.0),
                                        60.0)
                        w16 = jnp.exp(jnp.full((16,), a, jnp.float32))
                        a0 = a0 + w16 * kvv[i2, pl.ds(64, 16)]
                        a1 = a1 + w16 * kvv[i2, pl.ds(80, 16)]
                        a2 = a2 + w16 * kvv[i2, pl.ds(96, 16)]
                        a3 = a3 + w16 * kvv[i2, pl.ds(112, 16)]
                        return (a0, a1, a2, a3, ws + w16)

                    accs = lax.fori_loop(
                        i, seg_end, edge_body,
                        (zero16, zero16, zero16, zero16, zero16))
                    for f in range(4):
                        plsc.addupdate(acc_v.at[d, pl.ds(f * 16, 16)],
                                       accs[f])
                    plsc.addupdate(acc_v.at[d, pl.ds(64, 16)], accs[4])
                    return seg_end

                lax.while_loop(lambda i: i < i_hi, seg_body, i_lo)
                return carry

            lax.fori_loop(0, nch, chunk_body, 0)

            @pl.loop(0, _BD)
            def _norm(dd):
                sv = acc_v[dd, pl.ds(64, 16)]
                inv16 = 1.0 / (sv + 1e-16)
                for f in range(4):
                    q_v[dd, pl.ds(f * 16, 16)] = (
                        acc_v[dd, pl.ds(f * 16, 16)] * inv16)

            pltpu.sync_copy(q_v, out_h.at[pl.ds(d0s, _BD)])

    return ker(kv, q_scaled, src_s, dst_s, rowptr)


def _csr(ei):
    src, dst = ei[0], ei[1]
    order = jnp.argsort(dst)
    src_s = jnp.take(src, order).astype(jnp.int32)
    dst_s = jnp.take(dst, order).astype(jnp.int32)
    rowptr = jnp.searchsorted(
        dst_s, jnp.arange(_NPAD + 16, dtype=jnp.int32), side="left"
    ).astype(jnp.int32)
    pad = jnp.zeros((_EC + 8,), jnp.int32)
    return (jnp.concatenate([src_s, pad]), jnp.concatenate([dst_s, pad]),
            rowptr)


def kernel(x_transaction, merchant_ids, edge_index_tm, edge_index_mt,
           lin_tx_W, lin_tx_b, merch_emb, kW, kb, qW, qb, vW, vb, aW, ab,
           relA, relM, relP, skip, outW, outb):
    scale = 1.0 / np.sqrt(D)
    x0 = _mm_bias(x_transaction, lin_tx_W, lin_tx_b, _NPAD)
    ids_p = jnp.concatenate([merchant_ids.astype(jnp.int32),
                             jnp.zeros((_NPAD - NM,), jnp.int32)])
    x1 = jnp.take(merch_emb, ids_p, axis=0)
    x = {0: x0, 1: x1}
    edges = [(0, 1, 0, edge_index_tm), (1, 0, 1, edge_index_mt)]
    sizes = {0: NT, 1: NM}
    csr = {r: _csr(ei) for (_, _, r, ei) in edges}

    for l in range(L):
        q, kvrel = {}, {}
        for t in (0, 1):
            r_s, r_d = t, 1 - t
            cs = relP[l, r_d] * scale
            q[t], kvrel[r_s] = _proj(
                x[t], qW[l, t] * cs, qb[l, t] * cs, kW[l, t], kb[l, t],
                vW[l, t], vb[l, t], relA[l, r_s], relM[l, r_s])
        agg = {}
        for (s_t, d_t, r, ei) in edges:
            agg[d_t] = _edge_sc(kvrel[r], q[d_t], *csr[r])
        newx = {}
        for t in (0, 1):
            newx[t] = _update(agg[t], x[t], aW[l, t], ab[l, t],
                              skip[l, t:t + 1])
        x = newx

    outW_p = jnp.zeros((D, 128), jnp.float32).at[:, :OUT].set(outW)
    outb_p = jnp.zeros((128,), jnp.float32).at[:OUT].set(outb)
    out = _mm_bias(x[0], outW_p, outb_p, _NPAD)
    return out[:NT, :OUT]


# speedup vs baseline: 11.2490x; 1.1179x over previous
import functools

import jax
import jax.numpy as jnp
import numpy as np
from jax import lax
from jax.experimental import pallas as pl
from jax.experimental.pallas import tpu as pltpu
from jax.experimental.pallas import tpu_sc as plsc

NT = 50000
NM = 50000
D_IN = 128
D = 64
L = 4
OUT = 8
E = 800000

_ROWS = 2048

_NW = 32
_NPAD = 50176
_BDW = 1568
_NSUB = 4
_BD = 392
_EC = 128


def _mm_bias_kernel(x_ref, w_ref, b_ref, o_ref):
    o_ref[...] = (
        jnp.dot(x_ref[...], w_ref[...], preferred_element_type=jnp.float32)
        + b_ref[...]
    )


def _mm_bias(x, w, b, n_out):
    kdim = x.shape[1]
    m = w.shape[1]
    grid = (pl.cdiv(n_out, _ROWS),)
    return pl.pallas_call(
        _mm_bias_kernel,
        grid=grid,
        in_specs=[
            pl.BlockSpec((_ROWS, kdim), lambda i: (i, 0)),
            pl.BlockSpec((kdim, m), lambda i: (0, 0)),
            pl.BlockSpec((m,), lambda i: (0,)),
        ],
        out_specs=pl.BlockSpec((_ROWS, m), lambda i: (i, 0)),
        out_shape=jax.ShapeDtypeStruct((n_out, m), jnp.float32),
    )(x, w, b)


def _proj_kernel(x_ref, qw_ref, qb_ref, kw_ref, kb_ref, vw_ref, vb_ref,
                 ra_ref, rm_ref, q_ref, kv_ref):
    wk = jnp.dot(kw_ref[...], ra_ref[...], preferred_element_type=jnp.float32)
    wv = jnp.dot(vw_ref[...], rm_ref[...], preferred_element_type=jnp.float32)
    bk = jnp.dot(kb_ref[...].reshape(1, -1), ra_ref[...],
                 preferred_element_type=jnp.float32)
    bv = jnp.dot(vb_ref[...].reshape(1, -1), rm_ref[...],
                 preferred_element_type=jnp.float32)
    x = x_ref[...]
    q_ref[...] = jnp.dot(x, qw_ref[...], preferred_element_type=jnp.float32) + qb_ref[...]
    wkv = jnp.concatenate([wk, wv], axis=1)
    bkv = jnp.concatenate([bk, bv], axis=1)
    kv_ref[...] = jnp.dot(x, wkv, preferred_element_type=jnp.float32) + bkv


def _proj(x, qw, qb, kw, kb, vw, vb, ra, rm):
    grid = (pl.cdiv(_NPAD, _ROWS),)
    full = lambda i: (0, 0)
    vec = lambda i: (0,)
    row = lambda i: (i, 0)
    outs = [jax.ShapeDtypeStruct((_NPAD, D), jnp.float32),
            jax.ShapeDtypeStruct((_NPAD, 2 * D), jnp.float32)]
    return pl.pallas_call(
        _proj_kernel,
        grid=grid,
        in_specs=[
            pl.BlockSpec((_ROWS, D), row),
            pl.BlockSpec((D, D), full), pl.BlockSpec((D,), vec),
            pl.BlockSpec((D, D), full), pl.BlockSpec((D,), vec),
            pl.BlockSpec((D, D), full), pl.BlockSpec((D,), vec),
            pl.BlockSpec((D, D), full), pl.BlockSpec((D, D), full),
        ],
        out_specs=[pl.BlockSpec((_ROWS, D), row),
                   pl.BlockSpec((_ROWS, 2 * D), row)],
        out_shape=outs,
    )(x, qw, qb, kw, kb, vw, vb, ra, rm)


def _update_kernel(agg_ref, x_ref, aw_ref, ab_ref, skip_ref, o_ref):
    beta = jax.nn.sigmoid(skip_ref[0])
    o = (
        jnp.dot(jax.nn.gelu(agg_ref[...]), aw_ref[...],
                preferred_element_type=jnp.float32)
        + ab_ref[...]
    )
    o_ref[...] = jax.nn.relu(beta * o + (1.0 - beta) * x_ref[...])


def _update(agg, x, aw, ab, skip_scalar):
    n = agg.shape[0]
    grid = (pl.cdiv(n, _ROWS),)
    return pl.pallas_call(
        _update_kernel,
        grid=grid,
        in_specs=[
            pl.BlockSpec((_ROWS, D), lambda i: (i, 0)),
            pl.BlockSpec((_ROWS, D), lambda i: (i, 0)),
            pl.BlockSpec((D, D), lambda i: (0, 0)),
            pl.BlockSpec((D,), lambda i: (0,)),
            pl.BlockSpec((1,), lambda i: (0,)),
        ],
        out_specs=pl.BlockSpec((_ROWS, D), lambda i: (i, 0)),
        out_shape=jax.ShapeDtypeStruct((n, D), jnp.float32),
    )(agg, x, aw, ab, skip_scalar)


def _edge_sc(kv, q_scaled, ekey, rowptr):
    mesh = plsc.VectorSubcoreMesh(core_axis_name="c", subcore_axis_name="s")

    @functools.partial(
        pl.kernel,
        out_type=jax.ShapeDtypeStruct((_NPAD, D), jnp.float32),
        mesh=mesh,
        compiler_params=pltpu.CompilerParams(needs_layout_passes=False),
        scratch_types=[
            pltpu.VMEM((1584,), jnp.int32),
            pltpu.VMEM((_BD, D), jnp.float32),
            pltpu.VMEM((_BD, 80), jnp.float32),
            pltpu.VMEM((_EC,), jnp.int32),
            pltpu.VMEM((_EC + 16,), jnp.int32),
            pltpu.VMEM((_EC, 2 * D), jnp.float32),
        ],
    )
    def ker(kv_h, q_h, key_h, rp_h, out_h,
            rp_v, q_v, acc_v, src_v, key_v, kvv):
        wid = lax.axis_index("s") * 2 + lax.axis_index("c")
        w0 = wid * _BDW
        pltpu.sync_copy(rp_h.at[pl.ds(w0, 1584)], rp_v)

        def _sload(ref, idx):
            return ref[pl.ds(idx, 16)][0]

        @pl.loop(0, _NSUB)
        def _sub(sub):
            d0s = w0 + sub * _BD
            e_lo = _sload(rp_v, sub * _BD)
            e_hi = _sload(rp_v, (sub + 1) * _BD)

            @pl.loop(0, _BD)
            def _zero(dd):
                zero16 = jnp.zeros((16,), jnp.float32)
                for f in range(5):
                    acc_v[dd, pl.ds(f * 16, 16)] = zero16

            pltpu.sync_copy(q_h.at[pl.ds(d0s, _BD)], q_v)

            c0 = (e_lo // 8) * 8
            nch = (e_hi - c0 + _EC - 1) // _EC

            def chunk_body(j, carry):
                base = c0 + j * _EC
                pltpu.sync_copy(key_h.at[pl.ds(base, _EC)],
                                key_v.at[pl.ds(0, _EC)])
                for g in range(_EC // 16):
                    src_v[pl.ds(g * 16, 16)] = (
                        key_v[pl.ds(g * 16, 16)] & 0xFFFF)
                pltpu.sync_copy(kv_h.at[src_v], kvv)
                i_lo = jnp.maximum(e_lo - base, 0)
                i_hi = jnp.minimum(e_hi - base, _EC)

                def seg_body(i):
                    d = lax.shift_right_logical(_sload(key_v, i), 16) - d0s
                    seg_end = jnp.minimum(
                        _sload(rp_v, d + sub * _BD + 1) - base, i_hi)
                    vq = [q_v[d, pl.ds(f * 16, 16)] for f in range(4)]
                    zero16 = jnp.zeros((16,), jnp.float32)

                    def edge_body(i2, c2):
                        a0, a1, a2, a3, ws = c2
                        dot = kvv[i2, pl.ds(0, 16)] * vq[0]
                        for f in range(1, 4):
                            dot = dot + kvv[i2, pl.ds(f * 16, 16)] * vq[f]
                        a = jnp.minimum(jnp.maximum(jnp.sum(dot), -60.0),
                                        60.0)
                        w16 = jnp.exp(jnp.full((16,), a, jnp.float32))
                        a0 = a0 + w16 * kvv[i2, pl.ds(64, 16)]
                        a1 = a1 + w16 * kvv[i2, pl.ds(80, 16)]
                        a2 = a2 + w16 * kvv[i2, pl.ds(96, 16)]
                        a3 = a3 + w16 * kvv[i2, pl.ds(112, 16)]
                        return (a0, a1, a2, a3, ws + w16)

                    accs = lax.fori_loop(
                        i, seg_end, edge_body,
                        (zero16, zero16, zero16, zero16, zero16))
                    for f in range(4):
                        plsc.addupdate(acc_v.at[d, pl.ds(f * 16, 16)],
                                       accs[f])
                    plsc.addupdate(acc_v.at[d, pl.ds(64, 16)], accs[4])
                    return seg_end

                lax.while_loop(lambda i: i < i_hi, seg_body, i_lo)
                return carry

            lax.fori_loop(0, nch, chunk_body, 0)

            @pl.loop(0, _BD)
            def _norm(dd):
                sv = acc_v[dd, pl.ds(64, 16)]
                inv16 = 1.0 / (sv + 1e-16)
                for f in range(4):
                    q_v[dd, pl.ds(f * 16, 16)] = (
                        acc_v[dd, pl.ds(f * 16, 16)] * inv16)

            pltpu.sync_copy(q_v, out_h.at[pl.ds(d0s, _BD)])

    return ker(kv, q_scaled, ekey, rowptr)


def _csr(ei):
    src, dst = ei[0], ei[1]
    key = ((dst.astype(jnp.uint32) << 16) | src.astype(jnp.uint32))
    key_s = jnp.sort(key)
    bounds = jnp.arange(_NPAD + 16, dtype=jnp.uint32) << 16
    rowptr = jnp.searchsorted(key_s, bounds, side="left").astype(jnp.int32)
    key_i = lax.bitcast_convert_type(key_s, jnp.int32)
    pad = jnp.zeros((_EC + 8,), jnp.int32)
    return jnp.concatenate([key_i, pad]), rowptr


def kernel(x_transaction, merchant_ids, edge_index_tm, edge_index_mt,
           lin_tx_W, lin_tx_b, merch_emb, kW, kb, qW, qb, vW, vb, aW, ab,
           relA, relM, relP, skip, outW, outb):
    scale = 1.0 / np.sqrt(D)
    x0 = _mm_bias(x_transaction, lin_tx_W, lin_tx_b, _NPAD)
    ids_p = jnp.concatenate([merchant_ids.astype(jnp.int32),
                             jnp.zeros((_NPAD - NM,), jnp.int32)])
    x1 = jnp.take(merch_emb, ids_p, axis=0)
    x = {0: x0, 1: x1}
    edges = [(0, 1, 0, edge_index_tm), (1, 0, 1, edge_index_mt)]
    sizes = {0: NT, 1: NM}
    csr = {r: _csr(ei) for (_, _, r, ei) in edges}

    for l in range(L):
        q, kvrel = {}, {}
        for t in (0, 1):
            r_s, r_d = t, 1 - t
            cs = relP[l, r_d] * scale
            q[t], kvrel[r_s] = _proj(
                x[t], qW[l, t] * cs, qb[l, t] * cs, kW[l, t], kb[l, t],
                vW[l, t], vb[l, t], relA[l, r_s], relM[l, r_s])
        agg = {}
        for (s_t, d_t, r, ei) in edges:
            agg[d_t] = _edge_sc(kvrel[r], q[d_t], *csr[r])
        newx = {}
        for t in (0, 1):
            newx[t] = _update(agg[t], x[t], aW[l, t], ab[l, t],
                              skip[l, t:t + 1])
        x = newx

    outW_p = jnp.zeros((D, 128), jnp.float32).at[:, :OUT].set(outW)
    outb_p = jnp.zeros((128,), jnp.float32).at[:OUT].set(outb)
    out = _mm_bias(x[0], outW_p, outb_p, _NPAD)
    return out[:NT, :OUT]


# speedup vs baseline: 12.2808x; 1.0917x over previous
import functools

import jax
import jax.numpy as jnp
import numpy as np
from jax import lax
from jax.experimental import pallas as pl
from jax.experimental.pallas import tpu as pltpu
from jax.experimental.pallas import tpu_sc as plsc

NT = 50000
NM = 50000
D_IN = 128
D = 64
L = 4
OUT = 8
E = 800000

_ROWS = 2048

_NW = 32
_NPAD = 50176
_BDW = 1568
_NSUB = 4
_BD = 392
_EC = 128


def _mm_bias_kernel(x_ref, w_ref, b_ref, o_ref):
    o_ref[...] = (
        jnp.dot(x_ref[...], w_ref[...], preferred_element_type=jnp.float32)
        + b_ref[...]
    )


def _mm_bias(x, w, b, n_out):
    kdim = x.shape[1]
    m = w.shape[1]
    grid = (pl.cdiv(n_out, _ROWS),)
    return pl.pallas_call(
        _mm_bias_kernel,
        grid=grid,
        in_specs=[
            pl.BlockSpec((_ROWS, kdim), lambda i: (i, 0)),
            pl.BlockSpec((kdim, m), lambda i: (0, 0)),
            pl.BlockSpec((m,), lambda i: (0,)),
        ],
        out_specs=pl.BlockSpec((_ROWS, m), lambda i: (i, 0)),
        out_shape=jax.ShapeDtypeStruct((n_out, m), jnp.float32),
    )(x, w, b)


def _proj_kernel(x_ref, qw_ref, qb_ref, kw_ref, kb_ref, vw_ref, vb_ref,
                 ra_ref, rm_ref, q_ref, kv_ref):
    wk = jnp.dot(kw_ref[...], ra_ref[...], preferred_element_type=jnp.float32)
    wv = jnp.dot(vw_ref[...], rm_ref[...], preferred_element_type=jnp.float32)
    bk = jnp.dot(kb_ref[...].reshape(1, -1), ra_ref[...],
                 preferred_element_type=jnp.float32)
    bv = jnp.dot(vb_ref[...].reshape(1, -1), rm_ref[...],
                 preferred_element_type=jnp.float32)
    x = x_ref[...]
    q_ref[...] = jnp.dot(x, qw_ref[...], preferred_element_type=jnp.float32) + qb_ref[...]
    wkv = jnp.concatenate([wk, wv], axis=1)
    bkv = jnp.concatenate([bk, bv], axis=1)
    kv_ref[...] = jnp.dot(x, wkv, preferred_element_type=jnp.float32) + bkv


def _proj(x, qw, qb, kw, kb, vw, vb, ra, rm):
    grid = (pl.cdiv(_NPAD, _ROWS),)
    full = lambda i: (0, 0)
    vec = lambda i: (0,)
    row = lambda i: (i, 0)
    outs = [jax.ShapeDtypeStruct((_NPAD, D), jnp.float32),
            jax.ShapeDtypeStruct((_NPAD, 2 * D), jnp.float32)]
    return pl.pallas_call(
        _proj_kernel,
        grid=grid,
        in_specs=[
            pl.BlockSpec((_ROWS, D), row),
            pl.BlockSpec((D, D), full), pl.BlockSpec((D,), vec),
            pl.BlockSpec((D, D), full), pl.BlockSpec((D,), vec),
            pl.BlockSpec((D, D), full), pl.BlockSpec((D,), vec),
            pl.BlockSpec((D, D), full), pl.BlockSpec((D, D), full),
        ],
        out_specs=[pl.BlockSpec((_ROWS, D), row),
                   pl.BlockSpec((_ROWS, 2 * D), row)],
        out_shape=outs,
    )(x, qw, qb, kw, kb, vw, vb, ra, rm)


def _update_kernel(agg_ref, x_ref, aw_ref, ab_ref, skip_ref, o_ref):
    beta = jax.nn.sigmoid(skip_ref[0])
    o = (
        jnp.dot(jax.nn.gelu(agg_ref[...]), aw_ref[...],
                preferred_element_type=jnp.float32)
        + ab_ref[...]
    )
    o_ref[...] = jax.nn.relu(beta * o + (1.0 - beta) * x_ref[...])


def _update(agg, x, aw, ab, skip_scalar):
    n = agg.shape[0]
    grid = (pl.cdiv(n, _ROWS),)
    return pl.pallas_call(
        _update_kernel,
        grid=grid,
        in_specs=[
            pl.BlockSpec((_ROWS, D), lambda i: (i, 0)),
            pl.BlockSpec((_ROWS, D), lambda i: (i, 0)),
            pl.BlockSpec((D, D), lambda i: (0, 0)),
            pl.BlockSpec((D,), lambda i: (0,)),
            pl.BlockSpec((1,), lambda i: (0,)),
        ],
        out_specs=pl.BlockSpec((_ROWS, D), lambda i: (i, 0)),
        out_shape=jax.ShapeDtypeStruct((n, D), jnp.float32),
    )(agg, x, aw, ab, skip_scalar)


def _edge_sc(kv, q_scaled, ekey, rowptr):
    mesh = plsc.VectorSubcoreMesh(core_axis_name="c", subcore_axis_name="s")

    @functools.partial(
        pl.kernel,
        out_type=jax.ShapeDtypeStruct((_NPAD, D), jnp.float32),
        mesh=mesh,
        compiler_params=pltpu.CompilerParams(needs_layout_passes=False),
        scratch_types=[
            pltpu.VMEM((144,), jnp.int32),
            pltpu.VMEM((_BD, D), jnp.float32),
            pltpu.VMEM((_BD, 80), jnp.float32),
            pltpu.VMEM((_EC,), jnp.int32),
            pltpu.VMEM((_EC + 16,), jnp.int32),
            pltpu.VMEM((_EC, 2 * D), jnp.float32),
        ],
    )
    def ker(kv_h, q_h, key_h, rp_h, out_h,
            rp_v, q_v, acc_v, src_v, key_v, kvv):
        wid = lax.axis_index("s") * 2 + lax.axis_index("c")
        w0 = wid * _BDW
        pltpu.sync_copy(rp_h, rp_v.at[pl.ds(0, 136)])

        def _sload(ref, idx):
            return ref[pl.ds(idx, 16)][0]

        @pl.loop(0, _NSUB)
        def _sub(sub):
            d0s = w0 + sub * _BD
            e_lo = _sload(rp_v, wid * _NSUB + sub)
            e_hi = _sload(rp_v, wid * _NSUB + sub + 1)

            @pl.loop(0, _BD)
            def _zero(dd):
                zero16 = jnp.zeros((16,), jnp.float32)
                for f in range(5):
                    acc_v[dd, pl.ds(f * 16, 16)] = zero16

            pltpu.sync_copy(q_h.at[pl.ds(d0s, _BD)], q_v)

            c0 = (e_lo // 8) * 8
            nch = (e_hi - c0 + _EC - 1) // _EC

            def chunk_body(j, carry):
                base = c0 + j * _EC
                pltpu.sync_copy(key_h.at[pl.ds(base, _EC)],
                                key_v.at[pl.ds(0, _EC)])
                for g in range(_EC // 16):
                    src_v[pl.ds(g * 16, 16)] = (
                        key_v[pl.ds(g * 16, 16)] & 0xFFFF)
                pltpu.sync_copy(kv_h.at[src_v], kvv)
                i_lo = jnp.maximum(e_lo - base, 0)
                i_hi = jnp.minimum(e_hi - base, _EC)

                def seg_body(i):
                    dg = lax.shift_right_logical(_sload(key_v, i), 16)
                    d = dg - d0s
                    vq = [q_v[d, pl.ds(f * 16, 16)] for f in range(4)]
                    zero16 = jnp.zeros((16,), jnp.float32)

                    def edge_cond(c2):
                        i2 = c2[0]
                        same = lax.shift_right_logical(
                            _sload(key_v, i2), 16) == dg
                        return jnp.logical_and(i2 < i_hi, same)

                    def edge_body(c2):
                        i2, a0, a1, a2, a3, ws = c2
                        dot = kvv[i2, pl.ds(0, 16)] * vq[0]
                        for f in range(1, 4):
                            dot = dot + kvv[i2, pl.ds(f * 16, 16)] * vq[f]
                        a = jnp.minimum(jnp.maximum(jnp.sum(dot), -60.0),
                                        60.0)
                        w16 = jnp.exp(jnp.full((16,), a, jnp.float32))
                        a0 = a0 + w16 * kvv[i2, pl.ds(64, 16)]
                        a1 = a1 + w16 * kvv[i2, pl.ds(80, 16)]
                        a2 = a2 + w16 * kvv[i2, pl.ds(96, 16)]
                        a3 = a3 + w16 * kvv[i2, pl.ds(112, 16)]
                        return (i2 + 1, a0, a1, a2, a3, ws + w16)

                    accs = lax.while_loop(
                        edge_cond, edge_body,
                        (i, zero16, zero16, zero16, zero16, zero16))
                    for f in range(4):
                        plsc.addupdate(acc_v.at[d, pl.ds(f * 16, 16)],
                                       accs[f + 1])
                    plsc.addupdate(acc_v.at[d, pl.ds(64, 16)], accs[5])
                    return accs[0]

                lax.while_loop(lambda i: i < i_hi, seg_body, i_lo)
                return carry

            lax.fori_loop(0, nch, chunk_body, 0)

            @pl.loop(0, _BD)
            def _norm(dd):
                sv = acc_v[dd, pl.ds(64, 16)]
                inv16 = 1.0 / (sv + 1e-16)
                for f in range(4):
                    q_v[dd, pl.ds(f * 16, 16)] = (
                        acc_v[dd, pl.ds(f * 16, 16)] * inv16)

            pltpu.sync_copy(q_v, out_h.at[pl.ds(d0s, _BD)])

    return ker(kv, q_scaled, ekey, rowptr)


def _csr(ei):
    src, dst = ei[0], ei[1]
    key = ((dst.astype(jnp.uint32) << 16) | src.astype(jnp.uint32))
    key_s = jnp.sort(key)
    bounds = (jnp.arange(0, _NPAD + _BD, _BD, dtype=jnp.uint32)) << 16
    bptr = jnp.searchsorted(key_s, bounds, side="left").astype(jnp.int32)
    bptr = jnp.concatenate([bptr, jnp.full((7,), E, jnp.int32)])
    key_i = lax.bitcast_convert_type(key_s, jnp.int32)
    pad = jnp.zeros((_EC + 8,), jnp.int32)
    return jnp.concatenate([key_i, pad]), bptr


def kernel(x_transaction, merchant_ids, edge_index_tm, edge_index_mt,
           lin_tx_W, lin_tx_b, merch_emb, kW, kb, qW, qb, vW, vb, aW, ab,
           relA, relM, relP, skip, outW, outb):
    scale = 1.0 / np.sqrt(D)
    x0 = _mm_bias(x_transaction, lin_tx_W, lin_tx_b, _NPAD)
    ids_p = jnp.concatenate([merchant_ids.astype(jnp.int32),
                             jnp.zeros((_NPAD - NM,), jnp.int32)])
    x1 = jnp.take(merch_emb, ids_p, axis=0)
    x = {0: x0, 1: x1}
    edges = [(0, 1, 0, edge_index_tm), (1, 0, 1, edge_index_mt)]
    sizes = {0: NT, 1: NM}
    csr = {r: _csr(ei) for (_, _, r, ei) in edges}

    for l in range(L):
        q, kvrel = {}, {}
        for t in (0, 1):
            r_s, r_d = t, 1 - t
            cs = relP[l, r_d] * scale
            q[t], kvrel[r_s] = _proj(
                x[t], qW[l, t] * cs, qb[l, t] * cs, kW[l, t], kb[l, t],
                vW[l, t], vb[l, t], relA[l, r_s], relM[l, r_s])
        agg = {}
        for (s_t, d_t, r, ei) in edges:
            agg[d_t] = _edge_sc(kvrel[r], q[d_t], *csr[r])
        newx = {}
        for t in (0, 1):
            newx[t] = _update(agg[t], x[t], aW[l, t], ab[l, t],
                              skip[l, t:t + 1])
        x = newx

    outW_p = jnp.zeros((D, 128), jnp.float32).at[:, :OUT].set(outW)
    outb_p = jnp.zeros((128,), jnp.float32).at[:OUT].set(outb)
    out = _mm_bias(x[0], outW_p, outb_p, _NPAD)
    return out[:NT, :OUT]


# speedup vs baseline: 14.4833x; 1.1793x over previous
import functools

import jax
import jax.numpy as jnp
import numpy as np
from jax import lax
from jax.experimental import pallas as pl
from jax.experimental.pallas import tpu as pltpu
from jax.experimental.pallas import tpu_sc as plsc

NT = 50000
NM = 50000
D_IN = 128
D = 64
L = 4
OUT = 8
E = 800000

_ROWS = 2048

_NW = 32
_NPAD = 50176
_BDW = 1568
_NSUB = 4
_BD = 392
_EC = 112


def _mm_bias_kernel(x_ref, w_ref, b_ref, o_ref):
    o_ref[...] = (
        jnp.dot(x_ref[...], w_ref[...], preferred_element_type=jnp.float32)
        + b_ref[...]
    )


def _mm_bias(x, w, b, n_out):
    kdim = x.shape[1]
    m = w.shape[1]
    grid = (pl.cdiv(n_out, _ROWS),)
    return pl.pallas_call(
        _mm_bias_kernel,
        grid=grid,
        in_specs=[
            pl.BlockSpec((_ROWS, kdim), lambda i: (i, 0)),
            pl.BlockSpec((kdim, m), lambda i: (0, 0)),
            pl.BlockSpec((m,), lambda i: (0,)),
        ],
        out_specs=pl.BlockSpec((_ROWS, m), lambda i: (i, 0)),
        out_shape=jax.ShapeDtypeStruct((n_out, m), jnp.float32),
    )(x, w, b)


def _proj_kernel(x_ref, qw_ref, qb_ref, kw_ref, kb_ref, vw_ref, vb_ref,
                 ra_ref, rm_ref, q_ref, kv_ref):
    wk = jnp.dot(kw_ref[...], ra_ref[...], preferred_element_type=jnp.float32)
    wv = jnp.dot(vw_ref[...], rm_ref[...], preferred_element_type=jnp.float32)
    bk = jnp.dot(kb_ref[...].reshape(1, -1), ra_ref[...],
                 preferred_element_type=jnp.float32)
    bv = jnp.dot(vb_ref[...].reshape(1, -1), rm_ref[...],
                 preferred_element_type=jnp.float32)
    x = x_ref[...]
    q_ref[...] = jnp.dot(x, qw_ref[...], preferred_element_type=jnp.float32) + qb_ref[...]
    wkv = jnp.concatenate([wk, wv], axis=1)
    bkv = jnp.concatenate([bk, bv], axis=1)
    kv_ref[...] = jnp.dot(x, wkv, preferred_element_type=jnp.float32) + bkv


def _proj(x, qw, qb, kw, kb, vw, vb, ra, rm):
    grid = (pl.cdiv(_NPAD, _ROWS),)
    full = lambda i: (0, 0)
    vec = lambda i: (0,)
    row = lambda i: (i, 0)
    outs = [jax.ShapeDtypeStruct((_NPAD, D), jnp.float32),
            jax.ShapeDtypeStruct((_NPAD, 2 * D), jnp.float32)]
    return pl.pallas_call(
        _proj_kernel,
        grid=grid,
        in_specs=[
            pl.BlockSpec((_ROWS, D), row),
            pl.BlockSpec((D, D), full), pl.BlockSpec((D,), vec),
            pl.BlockSpec((D, D), full), pl.BlockSpec((D,), vec),
            pl.BlockSpec((D, D), full), pl.BlockSpec((D,), vec),
            pl.BlockSpec((D, D), full), pl.BlockSpec((D, D), full),
        ],
        out_specs=[pl.BlockSpec((_ROWS, D), row),
                   pl.BlockSpec((_ROWS, 2 * D), row)],
        out_shape=outs,
    )(x, qw, qb, kw, kb, vw, vb, ra, rm)


def _update_kernel(agg_ref, x_ref, aw_ref, ab_ref, skip_ref, o_ref):
    beta = jax.nn.sigmoid(skip_ref[0])
    o = (
        jnp.dot(jax.nn.gelu(agg_ref[...]), aw_ref[...],
                preferred_element_type=jnp.float32)
        + ab_ref[...]
    )
    o_ref[...] = jax.nn.relu(beta * o + (1.0 - beta) * x_ref[...])


def _update(agg, x, aw, ab, skip_scalar):
    n = agg.shape[0]
    grid = (pl.cdiv(n, _ROWS),)
    return pl.pallas_call(
        _update_kernel,
        grid=grid,
        in_specs=[
            pl.BlockSpec((_ROWS, D), lambda i: (i, 0)),
            pl.BlockSpec((_ROWS, D), lambda i: (i, 0)),
            pl.BlockSpec((D, D), lambda i: (0, 0)),
            pl.BlockSpec((D,), lambda i: (0,)),
            pl.BlockSpec((1,), lambda i: (0,)),
        ],
        out_specs=pl.BlockSpec((_ROWS, D), lambda i: (i, 0)),
        out_shape=jax.ShapeDtypeStruct((n, D), jnp.float32),
    )(agg, x, aw, ab, skip_scalar)


def _edge_sc(kv, q_scaled, ekey, rowptr):
    mesh = plsc.VectorSubcoreMesh(core_axis_name="c", subcore_axis_name="s")

    @functools.partial(
        pl.kernel,
        out_type=jax.ShapeDtypeStruct((_NPAD, D), jnp.float32),
        mesh=mesh,
        compiler_params=pltpu.CompilerParams(needs_layout_passes=False),
        scratch_types=[
            pltpu.VMEM((144,), jnp.int32),
            pltpu.VMEM((_BD, D), jnp.float32),
            pltpu.VMEM((_BD, 80), jnp.float32),
            pltpu.VMEM((_EC,), jnp.int32),
            pltpu.VMEM((_EC,), jnp.int32),
            pltpu.VMEM((_EC + 16,), jnp.int32),
            pltpu.VMEM((_EC + 16,), jnp.int32),
            pltpu.VMEM((_EC, 2 * D), jnp.float32),
            pltpu.VMEM((_EC, 2 * D), jnp.float32),
            pltpu.SemaphoreType.DMA,
            pltpu.SemaphoreType.DMA,
            pltpu.SemaphoreType.DMA,
            pltpu.SemaphoreType.DMA,
        ],
    )
    def ker(kv_h, q_h, key_h, rp_h, out_h,
            rp_v, q_v, acc_v, src_va, src_vb, key_va, key_vb,
            kv_va, kv_vb, sk0, sk1, sg0, sg1):
        wid = lax.axis_index("s") * 2 + lax.axis_index("c")
        w0 = wid * _BDW
        pltpu.sync_copy(rp_h, rp_v.at[pl.ds(0, 136)])

        def _sload(ref, idx):
            return ref[pl.ds(idx, 16)][0]

        @pl.loop(0, _NSUB)
        def _sub(sub):
            d0s = w0 + sub * _BD
            e_lo = _sload(rp_v, wid * _NSUB + sub)
            e_hi = _sload(rp_v, wid * _NSUB + sub + 1)

            @pl.loop(0, _BD)
            def _zero(dd):
                zero16 = jnp.zeros((16,), jnp.float32)
                for f in range(5):
                    acc_v[dd, pl.ds(f * 16, 16)] = zero16

            pltpu.sync_copy(q_h.at[pl.ds(d0s, _BD)], q_v)

            c0 = (e_lo // 8) * 8
            nch = (e_hi - c0 + _EC - 1) // _EC
            sks = (sk0, sk1)
            sgs = (sg0, sg1)
            srcs = (src_va, src_vb)
            keys = (key_va, key_vb)
            kvs = (kv_va, kv_vb)

            def _key_issue(j, b):
                pltpu.async_copy(key_h.at[pl.ds(c0 + j * _EC, _EC)],
                                 keys[b].at[pl.ds(0, _EC)], sks[b])

            def _key_wait(b):
                pltpu.make_async_copy(key_h.at[pl.ds(0, _EC)],
                                      keys[b].at[pl.ds(0, _EC)],
                                      sks[b]).wait()

            def _gather_issue(b):
                for g in range(_EC // 16):
                    srcs[b][pl.ds(g * 16, 16)] = (
                        keys[b][pl.ds(g * 16, 16)] & 0xFFFF)
                pltpu.async_copy(kv_h.at[srcs[b]], kvs[b], sgs[b])

            def _gather_wait(b):
                pltpu.make_async_copy(kv_h.at[srcs[b]], kvs[b],
                                      sgs[b]).wait()

            @pl.when(nch > 0)
            def _p0():
                _key_issue(0, 0)

            @pl.when(nch > 1)
            def _p1():
                _key_issue(1, 1)

            @pl.when(nch > 0)
            def _p2():
                _key_wait(0)
                _gather_issue(0)

            def _process(j, b, key_v, kvv):
                base = c0 + j * _EC
                i_lo = jnp.maximum(e_lo - base, 0)
                i_hi = jnp.minimum(e_hi - base, _EC)

                def seg_body(i):
                    dg = lax.shift_right_logical(_sload(key_v, i), 16)
                    d = dg - d0s
                    vq = [q_v[d, pl.ds(f * 16, 16)] for f in range(4)]
                    zero16 = jnp.zeros((16,), jnp.float32)

                    def edge_cond(c2):
                        i2 = c2[0]
                        same = lax.shift_right_logical(
                            _sload(key_v, i2), 16) == dg
                        return jnp.logical_and(i2 < i_hi, same)

                    def edge_body(c2):
                        i2, a0, a1, a2, a3, ws = c2
                        dot = kvv[i2, pl.ds(0, 16)] * vq[0]
                        for f in range(1, 4):
                            dot = dot + kvv[i2, pl.ds(f * 16, 16)] * vq[f]
                        a = jnp.minimum(jnp.maximum(jnp.sum(dot), -60.0),
                                        60.0)
                        w16 = jnp.exp(jnp.full((16,), a, jnp.float32))
                        a0 = a0 + w16 * kvv[i2, pl.ds(64, 16)]
                        a1 = a1 + w16 * kvv[i2, pl.ds(80, 16)]
                        a2 = a2 + w16 * kvv[i2, pl.ds(96, 16)]
                        a3 = a3 + w16 * kvv[i2, pl.ds(112, 16)]
                        return (i2 + 1, a0, a1, a2, a3, ws + w16)

                    accs = lax.while_loop(
                        edge_cond, edge_body,
                        (i, zero16, zero16, zero16, zero16, zero16))
                    for f in range(4):
                        plsc.addupdate(acc_v.at[d, pl.ds(f * 16, 16)],
                                       accs[f + 1])
                    plsc.addupdate(acc_v.at[d, pl.ds(64, 16)], accs[5])
                    return accs[0]

                lax.while_loop(lambda i: i < i_hi, seg_body, i_lo)

            def pair_body(jp, carry):
                for b in (0, 1):
                    j2 = jp * 2 + b

                    @pl.when(j2 < nch)
                    def _go(j2=j2, b=b):
                        @pl.when(j2 + 1 < nch)
                        def _prefetch():
                            _key_wait(1 - b)
                            _gather_issue(1 - b)

                        _gather_wait(b)
                        _process(j2, b, keys[b], kvs[b])

                        @pl.when(j2 + 2 < nch)
                        def _nextkey():
                            _key_issue(j2 + 2, b)

                return carry

            lax.fori_loop(0, (nch + 1) // 2, pair_body, 0)

            @pl.loop(0, _BD)
            def _norm(dd):
                sv = acc_v[dd, pl.ds(64, 16)]
                inv16 = 1.0 / (sv + 1e-16)
                for f in range(4):
                    q_v[dd, pl.ds(f * 16, 16)] = (
                        acc_v[dd, pl.ds(f * 16, 16)] * inv16)

            pltpu.sync_copy(q_v, out_h.at[pl.ds(d0s, _BD)])

    return ker(kv, q_scaled, ekey, rowptr)


def _csr(ei):
    src, dst = ei[0], ei[1]
    key = ((dst.astype(jnp.uint32) << 16) | src.astype(jnp.uint32))
    key_s = jnp.sort(key)
    bounds = (jnp.arange(0, _NPAD + _BD, _BD, dtype=jnp.uint32)) << 16
    bptr = jnp.searchsorted(key_s, bounds, side="left").astype(jnp.int32)
    bptr = jnp.concatenate([bptr, jnp.full((7,), E, jnp.int32)])
    key_i = lax.bitcast_convert_type(key_s, jnp.int32)
    pad = jnp.zeros((_EC + 8,), jnp.int32)
    return jnp.concatenate([key_i, pad]), bptr


def kernel(x_transaction, merchant_ids, edge_index_tm, edge_index_mt,
           lin_tx_W, lin_tx_b, merch_emb, kW, kb, qW, qb, vW, vb, aW, ab,
           relA, relM, relP, skip, outW, outb):
    scale = 1.0 / np.sqrt(D)
    x0 = _mm_bias(x_transaction, lin_tx_W, lin_tx_b, _NPAD)
    ids_p = jnp.concatenate([merchant_ids.astype(jnp.int32),
                             jnp.zeros((_NPAD - NM,), jnp.int32)])
    x1 = jnp.take(merch_emb, ids_p, axis=0)
    x = {0: x0, 1: x1}
    edges = [(0, 1, 0, edge_index_tm), (1, 0, 1, edge_index_mt)]
    sizes = {0: NT, 1: NM}
    csr = {r: _csr(ei) for (_, _, r, ei) in edges}

    for l in range(L):
        q, kvrel = {}, {}
        for t in (0, 1):
            r_s, r_d = t, 1 - t
            cs = relP[l, r_d] * scale
            q[t], kvrel[r_s] = _proj(
                x[t], qW[l, t] * cs, qb[l, t] * cs, kW[l, t], kb[l, t],
                vW[l, t], vb[l, t], relA[l, r_s], relM[l, r_s])
        agg = {}
        for (s_t, d_t, r, ei) in edges:
            agg[d_t] = _edge_sc(kvrel[r], q[d_t], *csr[r])
        newx = {}
        for t in (0, 1):
            newx[t] = _update(agg[t], x[t], aW[l, t], ab[l, t],
                              skip[l, t:t + 1])
        x = newx

    outW_p = jnp.zeros((D, 128), jnp.float32).at[:, :OUT].set(outW)
    outb_p = jnp.zeros((128,), jnp.float32).at[:OUT].set(outb)
    out = _mm_bias(x[0], outW_p, outb_p, _NPAD)
    return out[:NT, :OUT]
